# baseline (speedup 1.0000x reference)
import functools

import jax
import jax.numpy as jnp
from jax import lax
from jax.experimental import pallas as pl
from jax.experimental.pallas import tpu as pltpu
from jax.experimental.pallas import tpu_sc as plsc

NUM_EMB = 1000000
DIM = 64
BATCH = 16384
LANES = 16
LINE = 2 * DIM


def kernel(inputs, table):
    info = plsc.get_sparse_core_info()
    nc, ns = info.num_cores, info.num_subcores
    nw = nc * ns
    b_per_w = BATCH // nw

    mesh = plsc.VectorSubcoreMesh(core_axis_name="c", subcore_axis_name="s")

    @functools.partial(
        pl.kernel,
        out_type=jax.ShapeDtypeStruct((BATCH, 2, LINE), jnp.bfloat16),
        mesh=mesh,
        compiler_params=pltpu.CompilerParams(needs_layout_passes=False),
        scratch_types=[
            pltpu.VMEM((b_per_w,), jnp.int32),
            pltpu.SemaphoreType.DMA,
        ],
    )
    def emb_kernel(idx_hbm, tab_hbm, out_hbm, idx_s, sem):
        wid = lax.axis_index("s") * nc + lax.axis_index("c")
        base = pl.multiple_of(wid * b_per_w, b_per_w)

        pltpu.sync_copy(idx_hbm.at[pl.ds(base, b_per_w)], idx_s)

        def fire(j, _):
            vec = idx_s[pl.ds(j * LANES, LANES)]
            for k in range(LANES):
                p = pl.multiple_of((vec[k] // 4) * 2, 2)
                pltpu.async_copy(
                    tab_hbm.at[pl.ds(p, 2)],
                    out_hbm.at[base + j * LANES + k],
                    sem,
                )
            return ()

        lax.fori_loop(0, b_per_w // LANES, fire, ())

        pltpu.make_async_copy(
            out_hbm.at[pl.ds(0, b_per_w)],
            out_hbm.at[pl.ds(base, b_per_w)],
            sem,
        ).wait()

    tab_lines = table.astype(jnp.bfloat16).reshape(NUM_EMB // 2, LINE)
    lines = emb_kernel(inputs, tab_lines)
    quads = lines.reshape(BATCH, 4, DIM)
    return jnp.take_along_axis(
        quads, (inputs % 4)[:, None, None], axis=1
    ).reshape(BATCH, DIM)


# speedup vs baseline: 3.2629x; 3.2629x over previous
import functools

import jax
import jax.numpy as jnp
from jax import lax
from jax.experimental import pallas as pl
from jax.experimental.pallas import tpu as pltpu
from jax.experimental.pallas import tpu_sc as plsc

NUM_EMB = 1000000
DIM = 64
BATCH = 16384
LANES = 16


def kernel(inputs, table):
    info = plsc.get_sparse_core_info()
    nc, ns = info.num_cores, info.num_subcores
    nw = nc * ns
    b_per_w = BATCH // nw

    mesh = plsc.VectorSubcoreMesh(core_axis_name="c", subcore_axis_name="s")

    @functools.partial(
        pl.kernel,
        out_type=jax.ShapeDtypeStruct((BATCH, DIM), jnp.bfloat16),
        mesh=mesh,
        compiler_params=pltpu.CompilerParams(needs_layout_passes=False),
        scratch_types=[
            pltpu.VMEM((b_per_w,), jnp.int32),
            pltpu.VMEM((b_per_w, DIM), jnp.float32),
            pltpu.VMEM((b_per_w, DIM), jnp.bfloat16),
            pltpu.SemaphoreType.DMA,
        ],
    )
    def emb_kernel(idx_hbm, table_hbm, out_hbm, idx_s, rows_v, out_v, sem):
        wid = lax.axis_index("s") * nc + lax.axis_index("c")
        base = pl.multiple_of(wid * b_per_w, b_per_w)

        pltpu.sync_copy(idx_hbm.at[pl.ds(base, b_per_w)], idx_s)

        def fire(j, _):
            vec = idx_s[pl.ds(j * LANES, LANES)]
            for k in range(LANES):
                pltpu.async_copy(
                    table_hbm.at[vec[k]], rows_v.at[j * LANES + k], sem
                )
            return ()

        lax.fori_loop(0, b_per_w // LANES, fire, ())

        evens = jnp.arange(0, 2 * LANES, 2, dtype=jnp.int32)
        odds = evens + 1

        def body(r, _):
            pltpu.make_async_copy(table_hbm.at[0], rows_v.at[0], sem).wait()
            row = jnp.full((LANES,), r, dtype=jnp.int32)
            for half in range(DIM // (2 * LANES)):
                off = half * 2 * LANES
                a = plsc.load_gather(rows_v, [row, off + evens])
                b = plsc.load_gather(rows_v, [row, off + odds])
                packed = plsc.pack(a, b, format=plsc.PackFormat.INTERLEAVED)
                out_v[r, pl.ds(off, 2 * LANES)] = packed
            return ()

        lax.fori_loop(0, b_per_w, body, ())

        pltpu.sync_copy(out_v, out_hbm.at[pl.ds(base, b_per_w)])

    return emb_kernel(inputs, table)
